# baseline (speedup 1.0000x reference)
import dataclasses

import numpy as np

import jax
import jax.numpy as jnp
from jax import lax
from jax.experimental import pallas as pl
from jax.experimental.pallas import tpu as pltpu
from jax.experimental.pallas import tpu_sc as plsc

N = 10000
E = 320000
D = 128

NC = 2
NS = 16
L = 16
NW = NC * NS

C = 48
NSET = 4
CHUNKS_PER_W = 212
NCHUNK = NW * CHUNKS_PER_W
E_PAD = NCHUNK * C
NPAD = N + 8
HW = D // 2
ROWS_PER_TILE = 624
TAIL_BASE = NS * ROWS_PER_TILE

ROW_BLK = 400

_g = np.empty(D, np.int32)
for _c0 in range(0, D, 32):
    for _r in range(16):
        _g[_c0 + 2 * _r] = _c0 + _r
        _g[_c0 + 2 * _r + 1] = _c0 + 16 + _r
GPERM = _g


def _pq_body(ns_ref, wsrc_ref, wdst_ref, be_ref, p_ref, q_ref):
    ns = ns_ref[...]
    p_ref[...] = jnp.dot(ns, wsrc_ref[...], preferred_element_type=jnp.float32)
    q_ref[...] = (
        jnp.dot(ns, wdst_ref[...], preferred_element_type=jnp.float32)
        + be_ref[...]
    )


def _pq(node_states, w_src, w_dst, b_e_row):
    return pl.pallas_call(
        _pq_body,
        grid=(N // ROW_BLK,),
        in_specs=[
            pl.BlockSpec((ROW_BLK, D), lambda i: (i, 0)),
            pl.BlockSpec((D, D), lambda i: (0, 0)),
            pl.BlockSpec((D, D), lambda i: (0, 0)),
            pl.BlockSpec((1, D), lambda i: (0, 0)),
        ],
        out_specs=[
            pl.BlockSpec((ROW_BLK, D), lambda i: (i, 0)),
            pl.BlockSpec((ROW_BLK, D), lambda i: (i, 0)),
        ],
        out_shape=[
            jax.ShapeDtypeStruct((N, D), jnp.float32),
            jax.ShapeDtypeStruct((N, D), jnp.float32),
        ],
    )(node_states, w_src, w_dst, b_e_row)


def _upd_body(ns_ref, pp_ref, wt_ref, wb_ref, bu_ref, o_ref):
    aggr = pp_ref[0] + pp_ref[1]
    acc = jnp.dot(ns_ref[...], wt_ref[...], preferred_element_type=jnp.float32)
    acc = acc + jnp.dot(aggr, wb_ref[...], preferred_element_type=jnp.float32)
    o_ref[...] = jnp.maximum(acc + bu_ref[...], 0.0)


def _upd(node_states, partials, w_top, w_bot, b_u_row):
    return pl.pallas_call(
        _upd_body,
        grid=(N // ROW_BLK,),
        in_specs=[
            pl.BlockSpec((ROW_BLK, D), lambda i: (i, 0)),
            pl.BlockSpec((NC, ROW_BLK, D), lambda i: (0, i, 0)),
            pl.BlockSpec((D, D), lambda i: (0, 0)),
            pl.BlockSpec((D, D), lambda i: (0, 0)),
            pl.BlockSpec((1, D), lambda i: (0, 0)),
        ],
        out_specs=pl.BlockSpec((ROW_BLK, D), lambda i: (i, 0)),
        out_shape=jax.ShapeDtypeStruct((N, D), jnp.float32),
    )(node_states, partials, w_top, w_bot, b_u_row)


def _sc_edge_body(tt_hbm, eidx_hbm, out_hbm,
                  idx0, idx1, idx2, idx3, sdidx0, sdidx1, sdidx2, sdidx3,
                  g0, g1, g2, g3, m0, m1, m2, m3,
                  semg0, semg1, semg2, semg3, sems0, sems1, sems2, sems3,
                  semi0, semi1, semi2, semi3, aggr):
    cid = lax.axis_index("c")
    sid = lax.axis_index("s")
    wid = cid * NS + sid
    wbase = wid * CHUNKS_PER_W
    T = CHUNKS_PER_W

    sets = (
        (idx0, sdidx0, g0, m0, semg0, sems0, semi0),
        (idx1, sdidx1, g1, m1, semg1, sems1, semi1),
        (idx2, sdidx2, g2, m2, semg2, sems2, semi2),
        (idx3, sdidx3, g3, m3, semg3, sems3, semi3),
    )

    @pl.loop(0, C)
    def _zero_rows(r):
        for c0 in range(0, D, L):
            m0[r, pl.ds(c0, L)] = jnp.zeros((L,), jnp.float32)

    zbase = sid * ROWS_PER_TILE
    nfull = ROWS_PER_TILE // C
    @pl.loop(0, nfull)
    def _zcopy(k):
        pltpu.sync_copy(m0, aggr.at[pl.ds(zbase + k * C, C)])
    if ROWS_PER_TILE % C:
        pltpu.sync_copy(m0.at[pl.ds(0, ROWS_PER_TILE % C)],
                        aggr.at[pl.ds(zbase + nfull * C, ROWS_PER_TILE % C)])

    @pl.when(sid == NS - 1)
    def _ztail():
        pltpu.sync_copy(m0.at[pl.ds(0, NPAD - TAIL_BASE)],
                        aggr.at[pl.ds(TAIL_BASE, NPAD - TAIL_BASE)])

    plsc.subcore_barrier()

    def idx_start(t, st):
        ebase = (wbase + t) * (2 * C)
        pltpu.async_copy(eidx_hbm.at[pl.ds(ebase, 2 * C)], st[0].at[0], st[6])

    def idx_wait(st):
        pltpu.make_async_copy(eidx_hbm.at[pl.ds(0, 2 * C)], st[0].at[0],
                              st[6]).wait()

    def gather_start(st):
        pltpu.async_copy(tt_hbm.at[st[0].at[0]], st[2], st[4])

    def gather_wait(st):
        pltpu.make_async_copy(tt_hbm.at[st[0].at[0]], st[2], st[4]).wait()

    def scatter_start(st):
        pltpu.async_copy(st[3], aggr.at[st[1].at[0]], st[5], add=True)

    def scatter_wait(st):
        pltpu.make_async_copy(st[3], aggr.at[st[1].at[0]], st[5]).wait()

    def compute(st):
        g_c, m_c = st[2], st[3]

        @pl.loop(0, C)
        def _row(r):
            for ci in range(0, HW, L):
                av = plsc.bitcast(g_c[r, pl.ds(ci, L)], jnp.bfloat16)
                bv = plsc.bitcast(g_c[C + r, pl.ds(ci, L)], jnp.bfloat16)
                rm = jnp.maximum(av + bv, jnp.bfloat16(0))
                x, y = plsc.unpack(rm, format=plsc.PackFormat.INTERLEAVED)
                m_c[r, pl.ds(2 * ci, L)] = x
                m_c[r, pl.ds(2 * ci + L, L)] = y

    def make_sdidx(st):
        idx_c, sdidx_c = st[0], st[1]
        for k in range(0, C, L):
            sdidx_c[0, pl.ds(k, L)] = idx_c[0, pl.ds(C + k, L)] - NPAD

    idx_start(0, sets[0])
    idx_start(1, sets[1])
    idx_start(2, sets[2])
    idx_start(3, sets[3])
    idx_wait(sets[0])
    gather_start(sets[0])
    idx_wait(sets[1])
    gather_start(sets[1])
    idx_wait(sets[2])
    gather_start(sets[2])

    @pl.loop(0, T, step=NSET)
    def _trip(tbase):
        for k in range(NSET):
            t = tbase + k
            cur = sets[k]
            nxt3 = sets[(k + 3) % NSET]

            gather_wait(cur)

            @pl.when(t >= NSET)
            def _ws():
                scatter_wait(cur)

            make_sdidx(cur)

            @pl.when(t < T - NSET)
            def _pi():
                idx_start(t + NSET, cur)

            @pl.when(t < T - 3)
            def _ng():
                idx_wait(nxt3)
                gather_start(nxt3)

            compute(cur)
            scatter_start(cur)

    scatter_wait(sets[0])
    scatter_wait(sets[1])
    scatter_wait(sets[2])
    scatter_wait(sets[3])

    plsc.subcore_barrier()
    pltpu.sync_copy(aggr.at[pl.ds(zbase, ROWS_PER_TILE)],
                    out_hbm.at[cid].at[pl.ds(zbase, ROWS_PER_TILE)])

    @pl.when(sid == NS - 1)
    def _otail():
        pltpu.sync_copy(aggr.at[pl.ds(TAIL_BASE, N - TAIL_BASE)],
                        out_hbm.at[cid].at[pl.ds(TAIL_BASE, N - TAIL_BASE)])


@jax.jit
def _sc_edge(tt, eidx):
    mesh = plsc.VectorSubcoreMesh(
        core_axis_name="c", subcore_axis_name="s",
        num_cores=NC, num_subcores=NS)
    cp = pltpu.CompilerParams()
    if "needs_layout_passes" in pltpu.CompilerParams.__dataclass_fields__:
        cp = dataclasses.replace(cp, needs_layout_passes=False)
    if "use_tc_tiling_on_sc" in pltpu.CompilerParams.__dataclass_fields__:
        cp = dataclasses.replace(cp, use_tc_tiling_on_sc=False)
    k = pl.kernel(
        _sc_edge_body,
        out_type=jax.ShapeDtypeStruct((NC, N, D), jnp.float32),
        mesh=mesh,
        compiler_params=cp,
        scratch_types=[
            pltpu.VMEM((1, 2 * C), jnp.int32),
            pltpu.VMEM((1, 2 * C), jnp.int32),
            pltpu.VMEM((1, 2 * C), jnp.int32),
            pltpu.VMEM((1, 2 * C), jnp.int32),
            pltpu.VMEM((1, C), jnp.int32),
            pltpu.VMEM((1, C), jnp.int32),
            pltpu.VMEM((1, C), jnp.int32),
            pltpu.VMEM((1, C), jnp.int32),
            pltpu.VMEM((2 * C, HW), jnp.int32),
            pltpu.VMEM((2 * C, HW), jnp.int32),
            pltpu.VMEM((2 * C, HW), jnp.int32),
            pltpu.VMEM((2 * C, HW), jnp.int32),
            pltpu.VMEM((C, D), jnp.float32),
            pltpu.VMEM((C, D), jnp.float32),
            pltpu.VMEM((C, D), jnp.float32),
            pltpu.VMEM((C, D), jnp.float32),
            pltpu.SemaphoreType.DMA,
            pltpu.SemaphoreType.DMA,
            pltpu.SemaphoreType.DMA,
            pltpu.SemaphoreType.DMA,
            pltpu.SemaphoreType.DMA,
            pltpu.SemaphoreType.DMA,
            pltpu.SemaphoreType.DMA,
            pltpu.SemaphoreType.DMA,
            pltpu.SemaphoreType.DMA,
            pltpu.SemaphoreType.DMA,
            pltpu.SemaphoreType.DMA,
            pltpu.SemaphoreType.DMA,
            pltpu.VMEM_SHARED((NPAD, D), jnp.float32),
        ],
    )
    return k(tt, eidx)


def _pack_bf16(x):
    x16 = x.astype(jnp.bfloat16)
    return lax.bitcast_convert_type(x16.reshape(NPAD, HW, 2), jnp.int32)


def kernel(node_states, edges, W_e, b_e, W_u, b_u):
    src = edges[:, 0]
    dst = edges[:, 1]
    pad = jnp.full((E_PAD - E,), N, dtype=jnp.int32)
    src_p = jnp.concatenate([src, pad]).reshape(NCHUNK, C)
    dst_p = (jnp.concatenate([dst, pad]) + NPAD).reshape(NCHUNK, C)
    eidx = jnp.concatenate([src_p, dst_p], axis=1).reshape(-1)

    w_src = W_e[:D][:, GPERM]
    w_dst = W_e[D:][:, GPERM]
    b_perm = b_e[GPERM]
    p, q = _pq(node_states, w_src, w_dst, b_perm.reshape(1, D))
    p_pad = jnp.pad(p, ((0, NPAD - N), (0, 0)))
    q_pad = jnp.pad(q, ((0, NPAD - N), (0, 0)))
    tt = jnp.concatenate([_pack_bf16(p_pad), _pack_bf16(q_pad)], axis=0)

    partials = _sc_edge(tt, eidx)
    return _upd(node_states, partials, W_u[:D], W_u[D:], b_u.reshape(1, D))


# speedup vs baseline: 1.0204x; 1.0204x over previous
import dataclasses

import numpy as np

import jax
import jax.numpy as jnp
from jax import lax
from jax.experimental import pallas as pl
from jax.experimental.pallas import tpu as pltpu
from jax.experimental.pallas import tpu_sc as plsc

N = 10000
E = 320000
D = 128

NC = 2
NS = 16
L = 16
NW = NC * NS

C = 64
NSET = 3
CHUNKS_PER_W = 159
NCHUNK = NW * CHUNKS_PER_W
E_PAD = NCHUNK * C
NPAD = N + 8
HW = D // 2
ROWS_PER_TILE = 624
TAIL_BASE = NS * ROWS_PER_TILE

ROW_BLK = 400

_g = np.empty(D, np.int32)
for _c0 in range(0, D, 32):
    for _r in range(16):
        _g[_c0 + 2 * _r] = _c0 + _r
        _g[_c0 + 2 * _r + 1] = _c0 + 16 + _r
GPERM = _g


def _pq_body(ns_ref, wsrc_ref, wdst_ref, be_ref, p_ref, q_ref):
    ns = ns_ref[...]
    p_ref[...] = jnp.dot(ns, wsrc_ref[...], preferred_element_type=jnp.float32)
    q_ref[...] = (
        jnp.dot(ns, wdst_ref[...], preferred_element_type=jnp.float32)
        + be_ref[...]
    )


def _pq(node_states, w_src, w_dst, b_e_row):
    return pl.pallas_call(
        _pq_body,
        grid=(N // ROW_BLK,),
        in_specs=[
            pl.BlockSpec((ROW_BLK, D), lambda i: (i, 0)),
            pl.BlockSpec((D, D), lambda i: (0, 0)),
            pl.BlockSpec((D, D), lambda i: (0, 0)),
            pl.BlockSpec((1, D), lambda i: (0, 0)),
        ],
        out_specs=[
            pl.BlockSpec((ROW_BLK, D), lambda i: (i, 0)),
            pl.BlockSpec((ROW_BLK, D), lambda i: (i, 0)),
        ],
        out_shape=[
            jax.ShapeDtypeStruct((N, D), jnp.float32),
            jax.ShapeDtypeStruct((N, D), jnp.float32),
        ],
    )(node_states, w_src, w_dst, b_e_row)


def _upd_body(ns_ref, pp_ref, wt_ref, wb_ref, bu_ref, o_ref):
    aggr = pp_ref[0] + pp_ref[1]
    acc = jnp.dot(ns_ref[...], wt_ref[...], preferred_element_type=jnp.float32)
    acc = acc + jnp.dot(aggr, wb_ref[...], preferred_element_type=jnp.float32)
    o_ref[...] = jnp.maximum(acc + bu_ref[...], 0.0)


def _upd(node_states, partials, w_top, w_bot, b_u_row):
    return pl.pallas_call(
        _upd_body,
        grid=(N // ROW_BLK,),
        in_specs=[
            pl.BlockSpec((ROW_BLK, D), lambda i: (i, 0)),
            pl.BlockSpec((NC, ROW_BLK, D), lambda i: (0, i, 0)),
            pl.BlockSpec((D, D), lambda i: (0, 0)),
            pl.BlockSpec((D, D), lambda i: (0, 0)),
            pl.BlockSpec((1, D), lambda i: (0, 0)),
        ],
        out_specs=pl.BlockSpec((ROW_BLK, D), lambda i: (i, 0)),
        out_shape=jax.ShapeDtypeStruct((N, D), jnp.float32),
    )(node_states, partials, w_top, w_bot, b_u_row)


def _sc_edge_body(tt_hbm, eidx_hbm, out_hbm,
                  idx0, idx1, idx2, sdidx0, sdidx1, sdidx2,
                  g0, g1, g2, m0, m1, m2,
                  semg0, semg1, semg2, sems0, sems1, sems2,
                  semi0, semi1, semi2, aggr):
    cid = lax.axis_index("c")
    sid = lax.axis_index("s")
    wid = cid * NS + sid
    wbase = wid * CHUNKS_PER_W
    T = CHUNKS_PER_W

    sets = (
        (idx0, sdidx0, g0, m0, semg0, sems0, semi0),
        (idx1, sdidx1, g1, m1, semg1, sems1, semi1),
        (idx2, sdidx2, g2, m2, semg2, sems2, semi2),
    )

    @pl.loop(0, C)
    def _zero_rows(r):
        for c0 in range(0, D, L):
            m0[r, pl.ds(c0, L)] = jnp.zeros((L,), jnp.float32)

    zbase = sid * ROWS_PER_TILE
    nfull = ROWS_PER_TILE // C
    @pl.loop(0, nfull)
    def _zcopy(k):
        pltpu.sync_copy(m0, aggr.at[pl.ds(zbase + k * C, C)])
    if ROWS_PER_TILE % C:
        pltpu.sync_copy(m0.at[pl.ds(0, ROWS_PER_TILE % C)],
                        aggr.at[pl.ds(zbase + nfull * C, ROWS_PER_TILE % C)])

    @pl.when(sid == NS - 1)
    def _ztail():
        pltpu.sync_copy(m0.at[pl.ds(0, NPAD - TAIL_BASE)],
                        aggr.at[pl.ds(TAIL_BASE, NPAD - TAIL_BASE)])

    plsc.subcore_barrier()

    def idx_start(t, st):
        ebase = (wbase + t) * (2 * C)
        pltpu.async_copy(eidx_hbm.at[pl.ds(ebase, 2 * C)], st[0].at[0], st[6])

    def idx_wait(st):
        pltpu.make_async_copy(eidx_hbm.at[pl.ds(0, 2 * C)], st[0].at[0],
                              st[6]).wait()

    def gather_start(st):
        pltpu.async_copy(tt_hbm.at[st[0].at[0]], st[2], st[4])

    def gather_wait(st):
        pltpu.make_async_copy(tt_hbm.at[st[0].at[0]], st[2], st[4]).wait()

    def scatter_start(st):
        pltpu.async_copy(st[3], aggr.at[st[1].at[0]], st[5], add=True)

    def scatter_wait(st):
        pltpu.make_async_copy(st[3], aggr.at[st[1].at[0]], st[5]).wait()

    def compute(st):
        g_c, m_c = st[2], st[3]

        @pl.loop(0, C)
        def _row(r):
            for ci in range(0, HW, L):
                av = plsc.bitcast(g_c[r, pl.ds(ci, L)], jnp.bfloat16)
                bv = plsc.bitcast(g_c[C + r, pl.ds(ci, L)], jnp.bfloat16)
                rm = jnp.maximum(av + bv, jnp.bfloat16(0))
                x, y = plsc.unpack(rm, format=plsc.PackFormat.INTERLEAVED)
                m_c[r, pl.ds(2 * ci, L)] = x
                m_c[r, pl.ds(2 * ci + L, L)] = y

    def make_sdidx(st):
        idx_c, sdidx_c = st[0], st[1]
        for k in range(0, C, L):
            sdidx_c[0, pl.ds(k, L)] = idx_c[0, pl.ds(C + k, L)] - NPAD

    idx_start(0, sets[0])
    idx_start(1, sets[1])
    idx_start(2, sets[2])
    idx_wait(sets[0])
    gather_start(sets[0])
    idx_wait(sets[1])
    gather_start(sets[1])

    @pl.loop(0, T, step=NSET)
    def _trip(tbase):
        for k in range(NSET):
            t = tbase + k
            cur = sets[k]
            nxt2 = sets[(k + 2) % NSET]

            gather_wait(cur)

            @pl.when(t >= NSET)
            def _ws():
                scatter_wait(cur)

            make_sdidx(cur)

            @pl.when(t < T - NSET)
            def _pi():
                idx_start(t + NSET, cur)

            @pl.when(t < T - 2)
            def _ng():
                idx_wait(nxt2)
                gather_start(nxt2)

            compute(cur)
            scatter_start(cur)

    scatter_wait(sets[0])
    scatter_wait(sets[1])
    scatter_wait(sets[2])

    plsc.subcore_barrier()
    pltpu.sync_copy(aggr.at[pl.ds(zbase, ROWS_PER_TILE)],
                    out_hbm.at[cid].at[pl.ds(zbase, ROWS_PER_TILE)])

    @pl.when(sid == NS - 1)
    def _otail():
        pltpu.sync_copy(aggr.at[pl.ds(TAIL_BASE, N - TAIL_BASE)],
                        out_hbm.at[cid].at[pl.ds(TAIL_BASE, N - TAIL_BASE)])


@jax.jit
def _sc_edge(tt, eidx):
    mesh = plsc.VectorSubcoreMesh(
        core_axis_name="c", subcore_axis_name="s",
        num_cores=NC, num_subcores=NS)
    cp = pltpu.CompilerParams()
    if "needs_layout_passes" in pltpu.CompilerParams.__dataclass_fields__:
        cp = dataclasses.replace(cp, needs_layout_passes=False)
    if "use_tc_tiling_on_sc" in pltpu.CompilerParams.__dataclass_fields__:
        cp = dataclasses.replace(cp, use_tc_tiling_on_sc=False)
    k = pl.kernel(
        _sc_edge_body,
        out_type=jax.ShapeDtypeStruct((NC, N, D), jnp.float32),
        mesh=mesh,
        compiler_params=cp,
        scratch_types=[
            pltpu.VMEM((1, 2 * C), jnp.int32),
            pltpu.VMEM((1, 2 * C), jnp.int32),
            pltpu.VMEM((1, 2 * C), jnp.int32),
            pltpu.VMEM((1, C), jnp.int32),
            pltpu.VMEM((1, C), jnp.int32),
            pltpu.VMEM((1, C), jnp.int32),
            pltpu.VMEM((2 * C, HW), jnp.int32),
            pltpu.VMEM((2 * C, HW), jnp.int32),
            pltpu.VMEM((2 * C, HW), jnp.int32),
            pltpu.VMEM((C, D), jnp.float32),
            pltpu.VMEM((C, D), jnp.float32),
            pltpu.VMEM((C, D), jnp.float32),
            pltpu.SemaphoreType.DMA,
            pltpu.SemaphoreType.DMA,
            pltpu.SemaphoreType.DMA,
            pltpu.SemaphoreType.DMA,
            pltpu.SemaphoreType.DMA,
            pltpu.SemaphoreType.DMA,
            pltpu.SemaphoreType.DMA,
            pltpu.SemaphoreType.DMA,
            pltpu.SemaphoreType.DMA,
            pltpu.VMEM_SHARED((NPAD, D), jnp.float32),
        ],
    )
    return k(tt, eidx)


def _pack_bf16(x):
    x16 = x.astype(jnp.bfloat16)
    return lax.bitcast_convert_type(x16.reshape(NPAD, HW, 2), jnp.int32)


def kernel(node_states, edges, W_e, b_e, W_u, b_u):
    src = edges[:, 0]
    dst = edges[:, 1]
    pad = jnp.full((E_PAD - E,), N, dtype=jnp.int32)
    src_p = jnp.concatenate([src, pad]).reshape(NCHUNK, C)
    dst_p = (jnp.concatenate([dst, pad]) + NPAD).reshape(NCHUNK, C)
    eidx = jnp.concatenate([src_p, dst_p], axis=1).reshape(-1)

    w_src = W_e[:D][:, GPERM]
    w_dst = W_e[D:][:, GPERM]
    b_perm = b_e[GPERM]
    p, q = _pq(node_states, w_src, w_dst, b_perm.reshape(1, D))
    p_pad = jnp.pad(p, ((0, NPAD - N), (0, 0)))
    q_pad = jnp.pad(q, ((0, NPAD - N), (0, 0)))
    tt = jnp.concatenate([_pack_bf16(p_pad), _pack_bf16(q_pad)], axis=0)

    partials = _sc_edge(tt, eidx)
    return _upd(node_states, partials, W_u[:D], W_u[D:], b_u.reshape(1, D))


# speedup vs baseline: 1.1897x; 1.1659x over previous
import dataclasses

import numpy as np

import jax
import jax.numpy as jnp
from jax import lax
from jax.experimental import pallas as pl
from jax.experimental.pallas import tpu as pltpu
from jax.experimental.pallas import tpu_sc as plsc

N = 10000
E = 320000
D = 128

NC = 2
NS = 16
L = 16
NW = NC * NS

C = 64
NSET = 3
CHUNKS_PER_W = 159
NCHUNK = NW * CHUNKS_PER_W
E_PAD = NCHUNK * C
NPAD = N + 16
HW = D // 2
ROWS_PER_TILE = 624
TAIL_BASE = NS * ROWS_PER_TILE

PQ_BLK = 5008
UPD_BLK = 2000

_g = np.empty(D, np.int32)
for _c0 in range(0, D, 32):
    for _r in range(16):
        _g[_c0 + 2 * _r] = _c0 + _r
        _g[_c0 + 2 * _r + 1] = _c0 + 16 + _r
GPERM = _g


def _pq_body(ns_ref, wsrc_ref, wdst_ref, be_ref, p_ref, q_ref):
    ns = ns_ref[...]
    row = (pl.program_id(0) * PQ_BLK
           + lax.broadcasted_iota(jnp.int32, (PQ_BLK, 1), 0))
    live = row < N
    pv = jnp.dot(ns, wsrc_ref[...], preferred_element_type=jnp.float32)
    qv = (jnp.dot(ns, wdst_ref[...], preferred_element_type=jnp.float32)
          + be_ref[...])
    p_ref[...] = jnp.where(live, pv, 0.0).astype(jnp.bfloat16)
    q_ref[...] = jnp.where(live, qv, 0.0).astype(jnp.bfloat16)


def _pq(node_states, w_src, w_dst, b_e_row):
    return pl.pallas_call(
        _pq_body,
        grid=(NPAD // PQ_BLK,),
        in_specs=[
            pl.BlockSpec((PQ_BLK, D), lambda i: (i, 0)),
            pl.BlockSpec((D, D), lambda i: (0, 0)),
            pl.BlockSpec((D, D), lambda i: (0, 0)),
            pl.BlockSpec((1, D), lambda i: (0, 0)),
        ],
        out_specs=[
            pl.BlockSpec((PQ_BLK, D), lambda i: (i, 0)),
            pl.BlockSpec((PQ_BLK, D), lambda i: (i, 0)),
        ],
        out_shape=[
            jax.ShapeDtypeStruct((NPAD, D), jnp.bfloat16),
            jax.ShapeDtypeStruct((NPAD, D), jnp.bfloat16),
        ],
    )(node_states, w_src, w_dst, b_e_row)


def _upd_body(ns_ref, pp_ref, wt_ref, wb_ref, bu_ref, o_ref):
    aggr = pp_ref[0] + pp_ref[1]
    acc = jnp.dot(ns_ref[...], wt_ref[...], preferred_element_type=jnp.float32)
    acc = acc + jnp.dot(aggr, wb_ref[...], preferred_element_type=jnp.float32)
    o_ref[...] = jnp.maximum(acc + bu_ref[...], 0.0)


def _upd(node_states, partials, w_top, w_bot, b_u_row):
    return pl.pallas_call(
        _upd_body,
        grid=(N // UPD_BLK,),
        in_specs=[
            pl.BlockSpec((UPD_BLK, D), lambda i: (i, 0)),
            pl.BlockSpec((NC, UPD_BLK, D), lambda i: (0, i, 0)),
            pl.BlockSpec((D, D), lambda i: (0, 0)),
            pl.BlockSpec((D, D), lambda i: (0, 0)),
            pl.BlockSpec((1, D), lambda i: (0, 0)),
        ],
        out_specs=pl.BlockSpec((UPD_BLK, D), lambda i: (i, 0)),
        out_shape=jax.ShapeDtypeStruct((N, D), jnp.float32),
    )(node_states, partials, w_top, w_bot, b_u_row)


def _sc_edge_body(pt_hbm, qt_hbm, src_hbm, dst_hbm, out_hbm,
                  idx0, idx1, idx2, sdidx0, sdidx1, sdidx2,
                  g0, g1, g2, m0, m1, m2,
                  semg0, semg1, semg2, sems0, sems1, sems2,
                  semi0, semi1, semi2, aggr):
    cid = lax.axis_index("c")
    sid = lax.axis_index("s")
    wid = cid * NS + sid
    wbase = wid * CHUNKS_PER_W
    T = CHUNKS_PER_W

    sets = (
        (idx0, sdidx0, g0, m0, semg0, sems0, semi0),
        (idx1, sdidx1, g1, m1, semg1, sems1, semi1),
        (idx2, sdidx2, g2, m2, semg2, sems2, semi2),
    )

    @pl.loop(0, C)
    def _zero_rows(r):
        for c0 in range(0, D, L):
            m0[r, pl.ds(c0, L)] = jnp.zeros((L,), jnp.float32)

    zbase = sid * ROWS_PER_TILE
    nfull = ROWS_PER_TILE // C
    @pl.loop(0, nfull)
    def _zcopy(k):
        pltpu.sync_copy(m0, aggr.at[pl.ds(zbase + k * C, C)])
    if ROWS_PER_TILE % C:
        pltpu.sync_copy(m0.at[pl.ds(0, ROWS_PER_TILE % C)],
                        aggr.at[pl.ds(zbase + nfull * C, ROWS_PER_TILE % C)])

    @pl.when(sid == NS - 1)
    def _ztail():
        pltpu.sync_copy(m0.at[pl.ds(0, NPAD - TAIL_BASE)],
                        aggr.at[pl.ds(TAIL_BASE, NPAD - TAIL_BASE)])

    plsc.subcore_barrier()

    def idx_start(t, st):
        ebase = (wbase + t) * C
        pltpu.async_copy(src_hbm.at[pl.ds(ebase, C)], st[0].at[0], st[6])
        pltpu.async_copy(dst_hbm.at[pl.ds(ebase, C)], st[0].at[1], st[6])

    def idx_wait(st):
        pltpu.make_async_copy(src_hbm.at[pl.ds(0, C)], st[0].at[0],
                              st[6]).wait()
        pltpu.make_async_copy(dst_hbm.at[pl.ds(0, C)], st[0].at[1],
                              st[6]).wait()

    def gather_start(st):
        pltpu.async_copy(pt_hbm.at[st[0].at[0]], st[2].at[pl.ds(0, C)], st[4])
        pltpu.async_copy(qt_hbm.at[st[0].at[1]], st[2].at[pl.ds(C, C)], st[4])

    def gather_wait(st):
        pltpu.make_async_copy(pt_hbm.at[st[0].at[0]], st[2].at[pl.ds(0, C)],
                              st[4]).wait()
        pltpu.make_async_copy(qt_hbm.at[st[0].at[1]], st[2].at[pl.ds(C, C)],
                              st[4]).wait()

    def scatter_start(st):
        pltpu.async_copy(st[3], aggr.at[st[1].at[0]], st[5], add=True)

    def scatter_wait(st):
        pltpu.make_async_copy(st[3], aggr.at[st[1].at[0]], st[5]).wait()

    def compute(st):
        g_c, m_c = st[2], st[3]

        @pl.loop(0, C)
        def _row(r):
            for ci in range(0, HW, L):
                av = plsc.bitcast(g_c[r, pl.ds(ci, L)], jnp.bfloat16)
                bv = plsc.bitcast(g_c[C + r, pl.ds(ci, L)], jnp.bfloat16)
                rm = jnp.maximum(av + bv, jnp.bfloat16(0))
                x, y = plsc.unpack(rm, format=plsc.PackFormat.INTERLEAVED)
                m_c[r, pl.ds(2 * ci, L)] = x
                m_c[r, pl.ds(2 * ci + L, L)] = y

    def make_sdidx(st):
        idx_c, sdidx_c = st[0], st[1]
        for k in range(0, C, L):
            sdidx_c[0, pl.ds(k, L)] = idx_c[1, pl.ds(k, L)]

    idx_start(0, sets[0])
    idx_start(1, sets[1])
    idx_start(2, sets[2])
    idx_wait(sets[0])
    gather_start(sets[0])
    idx_wait(sets[1])
    gather_start(sets[1])

    @pl.loop(0, T, step=NSET)
    def _trip(tbase):
        for k in range(NSET):
            t = tbase + k
            cur = sets[k]
            nxt2 = sets[(k + 2) % NSET]

            gather_wait(cur)

            @pl.when(t >= NSET)
            def _ws():
                scatter_wait(cur)

            make_sdidx(cur)

            @pl.when(t < T - NSET)
            def _pi():
                idx_start(t + NSET, cur)

            @pl.when(t < T - 2)
            def _ng():
                idx_wait(nxt2)
                gather_start(nxt2)

            compute(cur)
            scatter_start(cur)

    scatter_wait(sets[0])
    scatter_wait(sets[1])
    scatter_wait(sets[2])

    plsc.subcore_barrier()
    pltpu.sync_copy(aggr.at[pl.ds(zbase, ROWS_PER_TILE)],
                    out_hbm.at[cid].at[pl.ds(zbase, ROWS_PER_TILE)])

    @pl.when(sid == NS - 1)
    def _otail():
        pltpu.sync_copy(aggr.at[pl.ds(TAIL_BASE, N - TAIL_BASE)],
                        out_hbm.at[cid].at[pl.ds(TAIL_BASE, N - TAIL_BASE)])


@jax.jit
def _sc_edge(pt, qt, src, dst):
    mesh = plsc.VectorSubcoreMesh(
        core_axis_name="c", subcore_axis_name="s",
        num_cores=NC, num_subcores=NS)
    cp = pltpu.CompilerParams()
    if "needs_layout_passes" in pltpu.CompilerParams.__dataclass_fields__:
        cp = dataclasses.replace(cp, needs_layout_passes=False)
    if "use_tc_tiling_on_sc" in pltpu.CompilerParams.__dataclass_fields__:
        cp = dataclasses.replace(cp, use_tc_tiling_on_sc=False)
    k = pl.kernel(
        _sc_edge_body,
        out_type=jax.ShapeDtypeStruct((NC, N, D), jnp.float32),
        mesh=mesh,
        compiler_params=cp,
        scratch_types=[
            pltpu.VMEM((2, C), jnp.int32),
            pltpu.VMEM((2, C), jnp.int32),
            pltpu.VMEM((2, C), jnp.int32),
            pltpu.VMEM((1, C), jnp.int32),
            pltpu.VMEM((1, C), jnp.int32),
            pltpu.VMEM((1, C), jnp.int32),
            pltpu.VMEM((2 * C, HW), jnp.int32),
            pltpu.VMEM((2 * C, HW), jnp.int32),
            pltpu.VMEM((2 * C, HW), jnp.int32),
            pltpu.VMEM((C, D), jnp.float32),
            pltpu.VMEM((C, D), jnp.float32),
            pltpu.VMEM((C, D), jnp.float32),
            pltpu.SemaphoreType.DMA,
            pltpu.SemaphoreType.DMA,
            pltpu.SemaphoreType.DMA,
            pltpu.SemaphoreType.DMA,
            pltpu.SemaphoreType.DMA,
            pltpu.SemaphoreType.DMA,
            pltpu.SemaphoreType.DMA,
            pltpu.SemaphoreType.DMA,
            pltpu.SemaphoreType.DMA,
            pltpu.VMEM_SHARED((NPAD, D), jnp.float32),
        ],
    )
    return k(pt, qt, src, dst)


def _pack_bf16(x16):
    return lax.bitcast_convert_type(x16.reshape(NPAD, HW, 2), jnp.int32)


def kernel(node_states, edges, W_e, b_e, W_u, b_u):
    pad = jnp.full((E_PAD - E,), N, dtype=jnp.int32)
    src_p = jnp.concatenate([edges[:, 0], pad])
    dst_p = jnp.concatenate([edges[:, 1], pad])

    w_src = W_e[:D][:, GPERM]
    w_dst = W_e[D:][:, GPERM]
    b_perm = b_e[GPERM]
    p16, q16 = _pq(node_states, w_src, w_dst, b_perm.reshape(1, D))

    partials = _sc_edge(_pack_bf16(p16), _pack_bf16(q16), src_p, dst_p)
    return _upd(node_states, partials, W_u[:D], W_u[D:], b_u.reshape(1, D))
